# baseline (speedup 1.0000x reference)
import functools
from functools import partial

import jax
import jax.numpy as jnp
from jax.experimental import pallas as pl
from jax.experimental.pallas import tpu as pltpu
from jax.sharding import PartitionSpec as P
from jax.experimental.shard_map import shard_map

_THETA = 10.0
_MAX_ITER = 30


def _sigmoid(x):
    return 1.0 / (1.0 + jnp.exp(-x))


def _logaddexp0(x):
    return jnp.maximum(x, 0.0) + jnp.log1p(jnp.exp(-jnp.abs(x)))


def _solver_kernel(axis_name,
                   a_ref, at_ref, b_ref, c_ref, u_ref,
                   x_ref, eta_ref,
                   q_out, q_in, s_out, s_in,
                   q_send, q_recv, s_send, s_recv):
    my_id = jax.lax.axis_index(axis_name)
    peer = 1 - my_id

    b = b_ref[...]
    c = c_ref[...]
    u = u_ref[...]
    theta_u = _THETA * u
    dtype_eps = float(jnp.finfo(jnp.float32).eps)
    btb = jnp.sum(b * b, axis=-1, keepdims=True)

    m_sz = a_ref.shape[0]
    n_loc = a_ref.shape[1]
    MC = 256
    NC = 512

    def A_mul_part(v):
        v16 = v.astype(jnp.bfloat16)
        acc = None
        for j in range(n_loc // NC):
            part = jax.lax.dot_general(
                v16[:, j * NC:(j + 1) * NC], at_ref[j * NC:(j + 1) * NC, :],
                (((1,), (0,)), ((), ())), preferred_element_type=jnp.float32)
            acc = part if acc is None else acc + part
        return acc

    def At_mul(w):
        w16 = w.astype(jnp.bfloat16)
        acc = None
        for i in range(m_sz // MC):
            part = jax.lax.dot_general(
                w16[:, i * MC:(i + 1) * MC], a_ref[i * MC:(i + 1) * MC, :],
                (((1,), (0,)), ((), ())), preferred_element_type=jnp.float32)
            acc = part if acc is None else acc + part
        return acc

    M = jnp.full((1, 1), _THETA, dtype=jnp.float32)
    beta_old = jnp.zeros((1, 1), dtype=jnp.float32)
    last_cond = jnp.zeros((1, 1), dtype=jnp.float32)
    eta = jnp.zeros((1, m_sz), dtype=jnp.float32)
    zeta = jnp.zeros((1, m_sz), dtype=jnp.float32)
    p_eta = jnp.zeros((1, n_loc), dtype=jnp.float32)
    p_zeta = jnp.zeros((1, n_loc), dtype=jnp.float32)
    x_final_pu = _sigmoid(-c * theta_u)

    def exchange(out_ref, in_ref, send, recv):
        rdma = pltpu.make_async_remote_copy(
            out_ref, in_ref, send, recv, device_id=peer,
            device_id_type=pl.DeviceIdType.LOGICAL)
        rdma.start()
        rdma.wait()

    def body(_, carry):
        (M, beta_old, last_cond, eta, zeta, p_eta, p_zeta, x_final_pu) = carry
        alpha = 0.5 / M + jnp.sqrt((0.25 / M + beta_old) / M)
        beta_new = beta_old + alpha
        tau = alpha / beta_new
        p_lam = p_eta + tau * (p_zeta - p_eta)
        neg_l = -(c - p_lam) * theta_u
        x_lam = _sigmoid(neg_l)
        q_out[...] = A_mul_part(u * x_lam)
        q = q_out[...] + q_in[...]
        grad = q - b
        zeta_new = zeta - alpha * grad
        eta_new = eta + tau * (zeta_new - eta)
        t = At_mul(grad)
        p_zeta_new = p_zeta - alpha * t
        p_eta_new = p_eta + tau * (p_zeta_new - p_eta)
        neg_e = -(c - p_eta_new) * theta_u
        s_loc = jnp.sum(_logaddexp0(neg_e) - _logaddexp0(neg_l),
                        axis=-1, keepdims=True)
        s_out[...] = jnp.broadcast_to(s_loc, s_out.shape)
        s_tot = s_out[0, 0] + s_in[0, 0]
        gap = (jnp.sum(q * q, axis=-1, keepdims=True) - btb) * (0.5 / M) + (
            s_tot / _THETA)
        cond = (gap <= dtype_eps).astype(jnp.float32)
        cond_b = cond > 0.5
        M = jnp.maximum(
            jnp.where(cond_b, jnp.where(last_cond > 0.5, M * 0.5, M), M * 2.0),
            dtype_eps)
        beta_old = jnp.where(cond_b, beta_new, beta_old)
        eta = jnp.where(cond_b, eta_new, eta)
        zeta = jnp.where(cond_b, zeta_new, zeta)
        p_eta = jnp.where(cond_b, p_eta_new, p_eta)
        p_zeta = jnp.where(cond_b, p_zeta_new, p_zeta)
        x_final_pu = jnp.where(cond_b, x_final_pu + tau * (x_lam - x_final_pu),
                               x_final_pu)
        return (M, beta_old, cond, eta, zeta, p_eta, p_zeta, x_final_pu)

    carry = (M, beta_old, last_cond, eta, zeta, p_eta, p_zeta, x_final_pu)
    carry = jax.lax.fori_loop(0, _MAX_ITER, body, carry)
    (_, _, _, eta, _, _, _, x_final_pu) = carry
    x_ref[...] = u * x_final_pu
    eta_ref[...] = eta


def _sharded_solve(a_l32, b, c_l, u_l, *, axis_name):
    a_l = a_l32.astype(jnp.bfloat16)
    at_l = a_l.T
    m = b.shape[1]
    n_loc = c_l.shape[1]
    x_l, eta = pl.pallas_call(
        partial(_solver_kernel, axis_name),
        out_shape=(jax.ShapeDtypeStruct((1, n_loc), jnp.float32),
                   jax.ShapeDtypeStruct((1, m), jnp.float32)),
        scratch_shapes=[
            pltpu.VMEM((1, m), jnp.float32),
            pltpu.VMEM((1, m), jnp.float32),
            pltpu.VMEM((1, 128), jnp.float32),
            pltpu.VMEM((1, 128), jnp.float32),
            pltpu.SemaphoreType.DMA,
            pltpu.SemaphoreType.DMA,
            pltpu.SemaphoreType.DMA,
            pltpu.SemaphoreType.DMA,
        ],
        compiler_params=pltpu.CompilerParams(
            vmem_limit_bytes=60 * 1024 * 1024,
            has_side_effects=True),
    )(a_l, at_l, b, c_l, u_l)
    return x_l, eta


@jax.jit
def kernel(A, b, c, u):
    devs = jax.devices()[:2]
    mesh = jax.sharding.Mesh(devs, ("x",))
    f = shard_map(
        partial(_sharded_solve, axis_name="x"),
        mesh=mesh,
        in_specs=(P(None, "x"), P(None, None), P(None, "x"), P(None, "x")),
        out_specs=(P(None, "x"), P(None, None)),
        check_rep=False)
    x_final, eta = f(A, b, c, u)
    return (x_final, eta)


# speedup vs baseline: 2.0329x; 2.0329x over previous
import functools

import jax
import jax.numpy as jnp
from jax.experimental import pallas as pl
from jax.experimental.pallas import tpu as pltpu

_THETA = 10.0
_MAX_ITER = 30
_XT = 6144
_NC = 512
_MC = 256


def _sigmoid(x):
    return 1.0 / (1.0 + jnp.exp(-x))


def _logaddexp0(x):
    return jnp.maximum(x, 0.0) + jnp.log1p(jnp.exp(-jnp.abs(x)))


def _solver_kernel(a_ref, b_ref, c_ref, u_ref, x_ref, eta_ref, at_ref):
    b = b_ref[...]
    c = c_ref[...]
    u = u_ref[...]
    theta_u = _THETA * u
    dtype_eps = float(jnp.finfo(jnp.float32).eps)
    btb = jnp.sum(b * b, axis=-1, keepdims=True)

    m_sz, n_sz = a_ref.shape

    for j in range(_XT // _NC):
        blk = a_ref[:, j * _NC:(j + 1) * _NC]
        at_ref[j * _NC:(j + 1) * _NC, :] = jnp.swapaxes(blk, 0, 1)

    def At_mul(w):
        w16 = w.astype(jnp.bfloat16)
        acc = None
        for i in range(m_sz // _MC):
            part = jax.lax.dot_general(
                w16[:, i * _MC:(i + 1) * _MC], a_ref[i * _MC:(i + 1) * _MC, :],
                (((1,), (0,)), ((), ())), preferred_element_type=jnp.float32)
            acc = part if acc is None else acc + part
        return acc

    def A_mul(v):
        v16 = v.astype(jnp.bfloat16)
        acc = None
        for j in range(_XT // _NC):
            part = jax.lax.dot_general(
                v16[:, j * _NC:(j + 1) * _NC], at_ref[j * _NC:(j + 1) * _NC, :],
                (((1,), (0,)), ((), ())), preferred_element_type=jnp.float32)
            acc = part if acc is None else acc + part
        for j in range(_XT // _NC, n_sz // _NC):
            part = jax.lax.dot_general(
                v16[:, j * _NC:(j + 1) * _NC], a_ref[:, j * _NC:(j + 1) * _NC],
                (((1,), (1,)), ((), ())), preferred_element_type=jnp.float32)
            acc = acc + part
        return acc

    M = jnp.full((1, 1), _THETA, dtype=jnp.float32)
    beta_old = jnp.zeros((1, 1), dtype=jnp.float32)
    last_cond = jnp.zeros((1, 1), dtype=jnp.float32)
    eta = jnp.zeros((1, m_sz), dtype=jnp.float32)
    zeta = jnp.zeros((1, m_sz), dtype=jnp.float32)
    p_eta = jnp.zeros((1, n_sz), dtype=jnp.float32)
    p_zeta = jnp.zeros((1, n_sz), dtype=jnp.float32)
    x_final_pu = _sigmoid(-c * theta_u)

    def body(_, carry):
        (M, beta_old, last_cond, eta, zeta, p_eta, p_zeta, x_final_pu) = carry
        alpha = 0.5 / M + jnp.sqrt((0.25 / M + beta_old) / M)
        beta_new = beta_old + alpha
        tau = alpha / beta_new
        p_lam = p_eta + tau * (p_zeta - p_eta)
        neg_l = -(c - p_lam) * theta_u
        x_lam = _sigmoid(neg_l)
        q = A_mul(u * x_lam)
        grad = q - b
        zeta_new = zeta - alpha * grad
        eta_new = eta + tau * (zeta_new - eta)
        t = At_mul(grad)
        p_zeta_new = p_zeta - alpha * t
        p_eta_new = p_eta + tau * (p_zeta_new - p_eta)
        neg_e = -(c - p_eta_new) * theta_u
        gap = (jnp.sum(q * q, axis=-1, keepdims=True) - btb) * (0.5 / M) + (
            jnp.sum(_logaddexp0(neg_e) - _logaddexp0(neg_l),
                    axis=-1, keepdims=True) / _THETA)
        cond = (gap <= dtype_eps).astype(jnp.float32)
        cond_b = cond > 0.5
        M = jnp.maximum(
            jnp.where(cond_b, jnp.where(last_cond > 0.5, M * 0.5, M), M * 2.0),
            dtype_eps)
        beta_old = jnp.where(cond_b, beta_new, beta_old)
        eta = jnp.where(cond_b, eta_new, eta)
        zeta = jnp.where(cond_b, zeta_new, zeta)
        p_eta = jnp.where(cond_b, p_eta_new, p_eta)
        p_zeta = jnp.where(cond_b, p_zeta_new, p_zeta)
        x_final_pu = jnp.where(cond_b, x_final_pu + tau * (x_lam - x_final_pu),
                               x_final_pu)
        return (M, beta_old, cond, eta, zeta, p_eta, p_zeta, x_final_pu)

    carry = (M, beta_old, last_cond, eta, zeta, p_eta, p_zeta, x_final_pu)
    carry = jax.lax.fori_loop(0, _MAX_ITER, body, carry)
    (_, _, _, eta, _, _, _, x_final_pu) = carry
    x_ref[...] = u * x_final_pu
    eta_ref[...] = eta


@jax.jit
def kernel(A, b, c, u):
    m, n = A.shape
    a_bf = A.astype(jnp.bfloat16)
    x_final, eta = pl.pallas_call(
        _solver_kernel,
        out_shape=(jax.ShapeDtypeStruct((1, n), jnp.float32),
                   jax.ShapeDtypeStruct((1, m), jnp.float32)),
        scratch_shapes=[pltpu.VMEM((_XT, m), jnp.bfloat16)],
        compiler_params=pltpu.CompilerParams(
            vmem_limit_bytes=100 * 1024 * 1024),
    )(a_bf, b, c, u)
    return (x_final, eta)


# speedup vs baseline: 2.1295x; 1.0475x over previous
import functools

import jax
import jax.numpy as jnp
from jax.experimental import pallas as pl
from jax.experimental.pallas import tpu as pltpu

_THETA = 10.0
_MAX_ITER = 30
_XT = 7168
_NC = 512
_MC = 256


def _sigmoid(x):
    return 1.0 / (1.0 + jnp.exp(-x))


def _logaddexp0(x):
    return jnp.maximum(x, 0.0) + jnp.log1p(jnp.exp(-jnp.abs(x)))


def _solver_kernel(a_ref, b_ref, c_ref, u_ref, x_ref, eta_ref, at_ref):
    b = b_ref[...]
    c = c_ref[...]
    u = u_ref[...]
    theta_u = _THETA * u
    dtype_eps = float(jnp.finfo(jnp.float32).eps)
    btb = jnp.sum(b * b, axis=-1, keepdims=True)

    m_sz, n_sz = a_ref.shape

    for j in range(_XT // _NC):
        blk = a_ref[:, j * _NC:(j + 1) * _NC]
        at_ref[j * _NC:(j + 1) * _NC, :] = jnp.swapaxes(blk, 0, 1)

    def At_mul(w):
        w16 = w.astype(jnp.bfloat16)
        acc = None
        for i in range(m_sz // _MC):
            part = jax.lax.dot_general(
                w16[:, i * _MC:(i + 1) * _MC], a_ref[i * _MC:(i + 1) * _MC, :],
                (((1,), (0,)), ((), ())), preferred_element_type=jnp.float32)
            acc = part if acc is None else acc + part
        return acc

    def A_mul(v):
        v16 = v.astype(jnp.bfloat16)
        acc = None
        for j in range(_XT // _NC):
            part = jax.lax.dot_general(
                v16[:, j * _NC:(j + 1) * _NC], at_ref[j * _NC:(j + 1) * _NC, :],
                (((1,), (0,)), ((), ())), preferred_element_type=jnp.float32)
            acc = part if acc is None else acc + part
        for j in range(_XT // _NC, n_sz // _NC):
            part = jax.lax.dot_general(
                v16[:, j * _NC:(j + 1) * _NC], a_ref[:, j * _NC:(j + 1) * _NC],
                (((1,), (1,)), ((), ())), preferred_element_type=jnp.float32)
            acc = acc + part
        return acc

    M = jnp.full((1, 1), _THETA, dtype=jnp.float32)
    beta_old = jnp.zeros((1, 1), dtype=jnp.float32)
    last_cond = jnp.zeros((1, 1), dtype=jnp.float32)
    eta = jnp.zeros((1, m_sz), dtype=jnp.float32)
    zeta = jnp.zeros((1, m_sz), dtype=jnp.float32)
    p_eta = jnp.zeros((1, n_sz), dtype=jnp.float32)
    p_zeta = jnp.zeros((1, n_sz), dtype=jnp.float32)
    x_final_pu = _sigmoid(-c * theta_u)

    def body(_, carry):
        (M, beta_old, last_cond, eta, zeta, p_eta, p_zeta, x_final_pu) = carry
        alpha = 0.5 / M + jnp.sqrt((0.25 / M + beta_old) / M)
        beta_new = beta_old + alpha
        tau = alpha / beta_new
        p_lam = p_eta + tau * (p_zeta - p_eta)
        neg_l = -(c - p_lam) * theta_u
        x_lam = _sigmoid(neg_l)
        q = A_mul(u * x_lam)
        grad = q - b
        zeta_new = zeta - alpha * grad
        eta_new = eta + tau * (zeta_new - eta)
        t = At_mul(grad)
        p_zeta_new = p_zeta - alpha * t
        p_eta_new = p_eta + tau * (p_zeta_new - p_eta)
        neg_e = -(c - p_eta_new) * theta_u
        gap = (jnp.sum(q * q, axis=-1, keepdims=True) - btb) * (0.5 / M) + (
            jnp.sum(_logaddexp0(neg_e) - _logaddexp0(neg_l),
                    axis=-1, keepdims=True) / _THETA)
        cond = (gap <= dtype_eps).astype(jnp.float32)
        cond_b = cond > 0.5
        M = jnp.maximum(
            jnp.where(cond_b, jnp.where(last_cond > 0.5, M * 0.5, M), M * 2.0),
            dtype_eps)
        beta_old = jnp.where(cond_b, beta_new, beta_old)
        eta = jnp.where(cond_b, eta_new, eta)
        zeta = jnp.where(cond_b, zeta_new, zeta)
        p_eta = jnp.where(cond_b, p_eta_new, p_eta)
        p_zeta = jnp.where(cond_b, p_zeta_new, p_zeta)
        x_final_pu = jnp.where(cond_b, x_final_pu + tau * (x_lam - x_final_pu),
                               x_final_pu)
        return (M, beta_old, cond, eta, zeta, p_eta, p_zeta, x_final_pu)

    carry = (M, beta_old, last_cond, eta, zeta, p_eta, p_zeta, x_final_pu)
    carry = jax.lax.fori_loop(0, _MAX_ITER, body, carry)
    (_, _, _, eta, _, _, _, x_final_pu) = carry
    x_ref[...] = u * x_final_pu
    eta_ref[...] = eta


@jax.jit
def kernel(A, b, c, u):
    m, n = A.shape
    a_bf = A.astype(jnp.bfloat16)
    x_final, eta = pl.pallas_call(
        _solver_kernel,
        out_shape=(jax.ShapeDtypeStruct((1, n), jnp.float32),
                   jax.ShapeDtypeStruct((1, m), jnp.float32)),
        scratch_shapes=[pltpu.VMEM((_XT, m), jnp.bfloat16)],
        compiler_params=pltpu.CompilerParams(
            vmem_limit_bytes=100 * 1024 * 1024),
    )(a_bf, b, c, u)
    return (x_final, eta)
